# baseline (speedup 1.0000x reference)
import functools

import jax
import jax.numpy as jnp
from jax import lax
from jax.experimental import pallas as pl
from jax.experimental.pallas import tpu as pltpu
from jax.experimental.pallas import tpu_sc as plsc

_B, _H, _W, _CIN = 4, 512, 512, 3
_GH, _GW, _GD, _GC = 16, 16, 8, 12
_COUT = _GC // (_CIN + 1)

_ROWS_PER_TEC = 64
_CHUNK = 8
_CPIX = _CHUNK * _W


def _conv_body(img_ref, gridt_ref, w_ref, b_ref, out_ref, outr_ref):
    f32 = jnp.float32
    rowf = jax.lax.broadcasted_iota(jnp.int32, (_H, 1), 0).astype(f32)
    gy = (rowf + 0.5) * (_GH / _H)
    wy1_full = gy - 0.5 - jnp.floor(gy - 0.5)
    for hc in range(_H // 16):
        fy = (hc - 1) // 2
        y0 = min(max(fy, 0), _GH - 1)
        y1 = min(max(fy + 1, 0), _GH - 1)
        wy1 = jax.lax.slice(wy1_full, (16 * hc, 0),
                            (16 * hc + 16, 1))
        g0 = gridt_ref[0, y0]
        g1 = gridt_ref[0, y1]
        outr_ref[0, pl.ds(16 * hc, 16), :] = (
            g0[None, :] * (1.0 - wy1) + g1[None, :] * wy1)

    padded = []
    for ci in range(_CIN):
        ich = img_ref[0, ci]
        hp = jnp.concatenate(
            [jnp.zeros((_H, 128), f32), ich, jnp.zeros((_H, 128), f32)], axis=1)
        vp = jnp.concatenate(
            [jnp.zeros((8, _W + 256), f32), hp, jnp.zeros((8, _W + 256), f32)],
            axis=0)
        padded.append(vp)
    for co in range(_CIN):
        acc = jnp.full((_H, _W), b_ref[co], f32)
        for dy in range(3):
            for dx in range(3):
                for ci in range(_CIN):
                    w = w_ref[dy, dx, ci, co]
                    acc = acc + w * jax.lax.slice(
                        padded[ci], (8 + dy - 1, 128 + dx - 1),
                        (8 + dy - 1 + _H, 128 + dx - 1 + _W))
        out_ref[0, co] = acc


def _run_conv(image_t, grid_t, W_conv, b_conv):
    return pl.pallas_call(
        _conv_body,
        grid=(_B,),
        in_specs=[
            pl.BlockSpec((1, _CIN, _H, _W), lambda b: (b, 0, 0, 0)),
            pl.BlockSpec((1, _GH, _GD * _GC * _GW), lambda b: (b, 0, 0)),
            pl.BlockSpec(memory_space=pltpu.SMEM),
            pl.BlockSpec(memory_space=pltpu.SMEM),
        ],
        out_specs=[
            pl.BlockSpec((1, _CIN, _H, _W), lambda b: (b, 0, 0, 0)),
            pl.BlockSpec((1, _H, _GD * _GC * _GW), lambda b: (b, 0, 0)),
        ],
        out_shape=[
            jax.ShapeDtypeStruct((_B, _CIN, _H, _W), jnp.float32),
            jax.ShapeDtypeStruct((_B, _H, _GD * _GC * _GW), jnp.float32),
        ],
    )(image_t, grid_t, W_conv, b_conv)


def _sc_body(rp_hbm, guide_hbm, img_hbm, out_hbm,
             rp_v, guide_v, img_v, out_v):
    f32 = jnp.float32
    i32 = jnp.int32
    cid = lax.axis_index("c")
    sid = lax.axis_index("s")
    wid = sid * 2 + cid
    batch = wid // 8
    rowbase = (wid % 8) * _ROWS_PER_TEC
    _NPIX = _H * _W
    _RROW = _GD * _GC * _GW

    pltpu.sync_copy(
        rp_hbm.at[pl.ds(batch * (_H * _RROW) + rowbase * _RROW,
                        _ROWS_PER_TEC * _RROW)], rp_v)

    def chunk_body(rc, carry):
        row0 = rowbase + rc * _CHUNK
        off = batch * _NPIX + row0 * _W
        pltpu.sync_copy(guide_hbm.at[pl.ds(off, _CPIX)], guide_v)
        for ci in range(_CIN):
            pltpu.sync_copy(
                img_hbm.at[pl.ds((batch * _CIN + ci - batch) * _NPIX + off,
                                 _CPIX)],
                img_v.at[pl.ds(ci * _CPIX, _CPIX)])

        def p_body(p, c2):
            i = lax.shift_right_logical(p, 5)
            j = lax.bitwise_and(p, 31)
            rl = rc * _CHUNK + i
            fxs = lax.shift_right_arithmetic(j - 1, 1)
            x0 = jnp.clip(fxs, 0, _GW - 1)
            x1 = jnp.clip(fxs + 1, 0, _GW - 1)
            colv = j * 16 + lax.iota(i32, 16)
            wx1 = (colv.astype(f32) + 0.5) * (1.0 / 32.0) - 0.5 - fxs.astype(f32)
            g = guide_v[pl.ds(p * 16, 16)]
            t = jnp.clip(g, 0.0, 1.0) * float(_GD) - 0.5
            ti = t.astype(i32)
            fzv = jnp.where(t < ti.astype(f32), ti - 1, ti)
            wz1 = t - fzv.astype(f32)
            z0 = jnp.clip(fzv, 0, _GD - 1)
            z1 = jnp.clip(fzv + 1, 0, _GD - 1)

            rbase = rl * _RROW
            wxs = ((1.0 - wx1), wx1)
            xbs = (rbase + x0, rbase + x1)
            wzs = ((1.0 - wz1), wz1)
            zbs = (z0 * (_GC * _GW), z1 * (_GC * _GW))

            coeff = [None] * _GC
            for b in range(2):
                for d in range(2):
                    w = wxs[b] * wzs[d]
                    base = zbs[d] + xbs[b]
                    for c in range(_GC):
                        gv = plsc.load_gather(rp_v, [base + c * _GW])
                        if coeff[c] is None:
                            coeff[c] = w * gv
                        else:
                            coeff[c] = coeff[c] + w * gv

            for co in range(_COUT):
                res = coeff[(_CIN + 1) * co + _CIN]
                for ci in range(_CIN):
                    imgv = img_v[pl.ds(ci * _CPIX + p * 16, 16)]
                    res = res + coeff[(_CIN + 1) * co + ci] * imgv
                out_v[pl.ds(co * _CPIX + p * 16, 16)] = res
            return c2

        lax.fori_loop(0, _CPIX // 16, p_body, 0)
        for co in range(_COUT):
            pltpu.sync_copy(
                out_v.at[pl.ds(co * _CPIX, _CPIX)],
                out_hbm.at[pl.ds((batch * _COUT + co - batch) * _NPIX + off,
                                 _CPIX)])
        return carry

    lax.fori_loop(0, _ROWS_PER_TEC // _CHUNK, chunk_body, 0)


def _run_sc(rp_flat, guide_flat, img_flat):
    mesh = plsc.VectorSubcoreMesh(core_axis_name="c", subcore_axis_name="s")
    f = functools.partial(
        pl.kernel,
        mesh=mesh,
        compiler_params=pltpu.CompilerParams(needs_layout_passes=False),
        out_type=jax.ShapeDtypeStruct((_B * _COUT * _H * _W,), jnp.float32),
        scratch_types=[
            pltpu.VMEM((_ROWS_PER_TEC * _GD * _GC * _GW,), jnp.float32),
            pltpu.VMEM((_CPIX,), jnp.float32),
            pltpu.VMEM((_CIN * _CPIX,), jnp.float32),
            pltpu.VMEM((_COUT * _CPIX,), jnp.float32),
        ],
    )(_sc_body)
    return f(rp_flat, guide_flat, img_flat)


def kernel(grid_th, guide_th, image_th, W_conv, b_conv):
    image_t = jnp.transpose(image_th, (0, 3, 1, 2))
    grid_t = jnp.transpose(grid_th, (0, 1, 3, 4, 2)).reshape(
        _B, _GH, _GD * _GC * _GW)
    conv_t, rp = _run_conv(image_t, grid_t, W_conv, b_conv)
    guide_flat = guide_th.reshape(-1)
    img_flat = conv_t.reshape(-1)
    out_flat = _run_sc(rp.reshape(-1), guide_flat, img_flat)
    out_t = out_flat.reshape(_B, _COUT, _H, _W)
    return jnp.transpose(out_t, (0, 2, 3, 1))


# speedup vs baseline: 3.0162x; 3.0162x over previous
import functools

import jax
import jax.numpy as jnp
from jax import lax
from jax.experimental import pallas as pl
from jax.experimental.pallas import tpu as pltpu
from jax.experimental.pallas import tpu_sc as plsc

_B, _H, _W, _CIN = 4, 512, 512, 3
_GH, _GW, _GD, _GC = 16, 16, 8, 12
_COUT = _GC // (_CIN + 1)

_ROWS_PER_TEC = 64
_CHUNK = 8
_CPIX = _CHUNK * _W


def _conv_body(img_ref, gridt_ref, w_ref, b_ref, out_ref, outr_ref):
    f32 = jnp.float32
    rowf = jax.lax.broadcasted_iota(jnp.int32, (_H, 1), 0).astype(f32)
    gy = (rowf + 0.5) * (_GH / _H)
    wy1_full = gy - 0.5 - jnp.floor(gy - 0.5)
    for hc in range(_H // 16):
        fy = (hc - 1) // 2
        y0 = min(max(fy, 0), _GH - 1)
        y1 = min(max(fy + 1, 0), _GH - 1)
        wy1 = jax.lax.slice(wy1_full, (16 * hc, 0),
                            (16 * hc + 16, 1))
        g0 = gridt_ref[0, y0]
        g1 = gridt_ref[0, y1]
        outr_ref[0, pl.ds(16 * hc, 16), :] = (
            g0[None, :] * (1.0 - wy1) + g1[None, :] * wy1)

    padded = []
    for ci in range(_CIN):
        ich = img_ref[0, ci]
        hp = jnp.concatenate(
            [jnp.zeros((_H, 128), f32), ich, jnp.zeros((_H, 128), f32)], axis=1)
        vp = jnp.concatenate(
            [jnp.zeros((8, _W + 256), f32), hp, jnp.zeros((8, _W + 256), f32)],
            axis=0)
        padded.append(vp)
    for co in range(_CIN):
        acc = jnp.full((_H, _W), b_ref[co], f32)
        for dy in range(3):
            for dx in range(3):
                for ci in range(_CIN):
                    w = w_ref[dy, dx, ci, co]
                    acc = acc + w * jax.lax.slice(
                        padded[ci], (8 + dy - 1, 128 + dx - 1),
                        (8 + dy - 1 + _H, 128 + dx - 1 + _W))
        out_ref[0, co] = acc


def _run_conv(image_t, grid_t, W_conv, b_conv):
    return pl.pallas_call(
        _conv_body,
        grid=(_B,),
        in_specs=[
            pl.BlockSpec((1, _CIN, _H, _W), lambda b: (b, 0, 0, 0)),
            pl.BlockSpec((1, _GH, _GD * _GC * _GW), lambda b: (b, 0, 0)),
            pl.BlockSpec(memory_space=pltpu.SMEM),
            pl.BlockSpec(memory_space=pltpu.SMEM),
        ],
        out_specs=[
            pl.BlockSpec((1, _CIN, _H, _W), lambda b: (b, 0, 0, 0)),
            pl.BlockSpec((1, _H, _GD * _GC * _GW), lambda b: (b, 0, 0)),
        ],
        out_shape=[
            jax.ShapeDtypeStruct((_B, _CIN, _H, _W), jnp.float32),
            jax.ShapeDtypeStruct((_B, _H, _GD * _GC * _GW), jnp.float32),
        ],
    )(image_t, grid_t, W_conv, b_conv)


def _sc_body(rp_hbm, guide_hbm, img_hbm, out_hbm,
             rp_v, guide_v, img_v, out_v):
    f32 = jnp.float32
    i32 = jnp.int32
    cid = lax.axis_index("c")
    sid = lax.axis_index("s")
    wid = sid * 2 + cid
    batch = wid // 8
    rowbase = (wid % 8) * _ROWS_PER_TEC
    _NPIX = _H * _W
    _RROW = _GD * _GC * _GW

    pltpu.sync_copy(
        rp_hbm.at[pl.ds(batch * (_H * _RROW) + rowbase * _RROW,
                        _ROWS_PER_TEC * _RROW)], rp_v)

    def chunk_body(rc, carry):
        row0 = rowbase + rc * _CHUNK
        off = batch * _NPIX + row0 * _W
        pltpu.sync_copy(guide_hbm.at[pl.ds(off, _CPIX)], guide_v)
        for ci in range(_CIN):
            pltpu.sync_copy(
                img_hbm.at[pl.ds((batch * _CIN + ci - batch) * _NPIX + off,
                                 _CPIX)],
                img_v.at[pl.ds(ci * _CPIX, _CPIX)])

        def p_body(p, c2):
            i = lax.shift_right_logical(p, 5)
            j = lax.bitwise_and(p, 31)
            rl = rc * _CHUNK + i
            fxs = lax.shift_right_arithmetic(j - 1, 1)
            x0 = jnp.clip(fxs, 0, _GW - 1)
            x1 = jnp.clip(fxs + 1, 0, _GW - 1)
            colv = j * 16 + lax.iota(i32, 16)
            wx1 = (colv.astype(f32) + 0.5) * (1.0 / 32.0) - 0.5 - fxs.astype(f32)
            g = guide_v[pl.ds(p * 16, 16)]
            t = jnp.clip(g, 0.0, 1.0) * float(_GD) - 0.5
            ti = t.astype(i32)
            fzv = jnp.where(t < ti.astype(f32), ti - 1, ti)
            wz1 = t - fzv.astype(f32)
            z0 = jnp.clip(fzv, 0, _GD - 1)
            z1 = jnp.clip(fzv + 1, 0, _GD - 1)

            rbase = rl * _RROW
            wxs = ((1.0 - wx1), wx1)
            xbs = (rbase + x0 * _GD, rbase + x1 * _GD)
            wzs = ((1.0 - wz1), wz1)

            coeff = [None] * _GC
            for b in range(2):
                for d in range(2):
                    w = wxs[b] * wzs[d]
                    base = (z0 if d == 0 else z1) + xbs[b]
                    for c in range(_GC):
                        gv = plsc.load_gather(rp_v, [base + c * (_GW * _GD)])
                        if coeff[c] is None:
                            coeff[c] = w * gv
                        else:
                            coeff[c] = coeff[c] + w * gv

            for co in range(_COUT):
                res = coeff[(_CIN + 1) * co + _CIN]
                for ci in range(_CIN):
                    imgv = img_v[pl.ds(ci * _CPIX + p * 16, 16)]
                    res = res + coeff[(_CIN + 1) * co + ci] * imgv
                out_v[pl.ds(co * _CPIX + p * 16, 16)] = res
            return c2

        lax.fori_loop(0, _CPIX // 16, p_body, 0)
        for co in range(_COUT):
            pltpu.sync_copy(
                out_v.at[pl.ds(co * _CPIX, _CPIX)],
                out_hbm.at[pl.ds((batch * _COUT + co - batch) * _NPIX + off,
                                 _CPIX)])
        return carry

    lax.fori_loop(0, _ROWS_PER_TEC // _CHUNK, chunk_body, 0)


def _run_sc(rp_flat, guide_flat, img_flat):
    mesh = plsc.VectorSubcoreMesh(core_axis_name="c", subcore_axis_name="s")
    f = functools.partial(
        pl.kernel,
        mesh=mesh,
        compiler_params=pltpu.CompilerParams(needs_layout_passes=False),
        out_type=jax.ShapeDtypeStruct((_B * _COUT * _H * _W,), jnp.float32),
        scratch_types=[
            pltpu.VMEM((_ROWS_PER_TEC * _GD * _GC * _GW,), jnp.float32),
            pltpu.VMEM((_CPIX,), jnp.float32),
            pltpu.VMEM((_CIN * _CPIX,), jnp.float32),
            pltpu.VMEM((_COUT * _CPIX,), jnp.float32),
        ],
    )(_sc_body)
    return f(rp_flat, guide_flat, img_flat)


def kernel(grid_th, guide_th, image_th, W_conv, b_conv):
    image_t = jnp.transpose(image_th, (0, 3, 1, 2))
    grid_t = jnp.transpose(grid_th, (0, 1, 4, 2, 3)).reshape(
        _B, _GH, _GD * _GC * _GW)
    conv_t, rp = _run_conv(image_t, grid_t, W_conv, b_conv)
    guide_flat = guide_th.reshape(-1)
    img_flat = conv_t.reshape(-1)
    out_flat = _run_sc(rp.reshape(-1), guide_flat, img_flat)
    out_t = out_flat.reshape(_B, _COUT, _H, _W)
    return jnp.transpose(out_t, (0, 2, 3, 1))


# speedup vs baseline: 3.3361x; 1.1061x over previous
import functools

import jax
import jax.numpy as jnp
from jax import lax
from jax.experimental import pallas as pl
from jax.experimental.pallas import tpu as pltpu
from jax.experimental.pallas import tpu_sc as plsc

_B, _H, _W, _CIN = 4, 512, 512, 3
_GH, _GW, _GD, _GC = 16, 16, 8, 12
_COUT = _GC // (_CIN + 1)

_ROWS_PER_TEC = 64
_CHUNK = 8
_CPIX = _CHUNK * _W


def _conv_body(img_ref, gridt_ref, w_ref, b_ref, out_ref, outr_ref):
    f32 = jnp.float32
    rowf = jax.lax.broadcasted_iota(jnp.int32, (_H, 1), 0).astype(f32)
    gy = (rowf + 0.5) * (_GH / _H)
    wy1_full = gy - 0.5 - jnp.floor(gy - 0.5)
    for hc in range(_H // 16):
        fy = (hc - 1) // 2
        y0 = min(max(fy, 0), _GH - 1)
        y1 = min(max(fy + 1, 0), _GH - 1)
        wy1 = jax.lax.slice(wy1_full, (16 * hc, 0),
                            (16 * hc + 16, 1))
        g0 = gridt_ref[0, y0]
        g1 = gridt_ref[0, y1]
        outr_ref[0, pl.ds(16 * hc, 16), :] = (
            g0[None, :] * (1.0 - wy1) + g1[None, :] * wy1)

    padded = []
    for ci in range(_CIN):
        ich = img_ref[0, ci]
        hp = jnp.concatenate(
            [jnp.zeros((_H, 128), f32), ich, jnp.zeros((_H, 128), f32)], axis=1)
        vp = jnp.concatenate(
            [jnp.zeros((8, _W + 256), f32), hp, jnp.zeros((8, _W + 256), f32)],
            axis=0)
        padded.append(vp)
    for co in range(_CIN):
        acc = jnp.full((_H, _W), b_ref[co], f32)
        for dy in range(3):
            for dx in range(3):
                for ci in range(_CIN):
                    w = w_ref[dy, dx, ci, co]
                    acc = acc + w * jax.lax.slice(
                        padded[ci], (8 + dy - 1, 128 + dx - 1),
                        (8 + dy - 1 + _H, 128 + dx - 1 + _W))
        out_ref[0, co] = acc


def _run_conv(image_t, grid_t, W_conv, b_conv):
    return pl.pallas_call(
        _conv_body,
        grid=(_B,),
        in_specs=[
            pl.BlockSpec((1, _CIN, _H, _W), lambda b: (b, 0, 0, 0)),
            pl.BlockSpec((1, _GH, _GD * _GC * _GW), lambda b: (b, 0, 0)),
            pl.BlockSpec(memory_space=pltpu.SMEM),
            pl.BlockSpec(memory_space=pltpu.SMEM),
        ],
        out_specs=[
            pl.BlockSpec((1, _CIN, _H, _W), lambda b: (b, 0, 0, 0)),
            pl.BlockSpec((1, _H, _GD * _GC * _GW), lambda b: (b, 0, 0)),
        ],
        out_shape=[
            jax.ShapeDtypeStruct((_B, _CIN, _H, _W), jnp.float32),
            jax.ShapeDtypeStruct((_B, _H, _GD * _GC * _GW), jnp.float32),
        ],
    )(image_t, grid_t, W_conv, b_conv)


def _sc_body(rp_hbm, guide_hbm, img_hbm, out_hbm,
             rp_v, guide_v, img_v, out_v):
    f32 = jnp.float32
    i32 = jnp.int32
    cid = lax.axis_index("c")
    sid = lax.axis_index("s")
    wid = sid * 2 + cid
    batch = wid // 8
    rowbase = (wid % 8) * _ROWS_PER_TEC
    _NPIX = _H * _W
    _RROW = _GD * _GC * _GW

    pltpu.sync_copy(
        rp_hbm.at[pl.ds(batch * (_H * _RROW) + rowbase * _RROW,
                        _ROWS_PER_TEC * _RROW)], rp_v)

    def chunk_body(rc, carry):
        row0 = rowbase + rc * _CHUNK
        off = batch * _NPIX + row0 * _W
        pltpu.sync_copy(guide_hbm.at[pl.ds(off, _CPIX)], guide_v)
        for ci in range(_CIN):
            pltpu.sync_copy(
                img_hbm.at[pl.ds((batch * _CIN + ci - batch) * _NPIX + off,
                                 _CPIX)],
                img_v.at[pl.ds(ci * _CPIX, _CPIX)])

        @plsc.parallel_loop(0, _CPIX // 16, unroll=2)
        def p_body(p):
            i = lax.shift_right_logical(p, 5)
            j = lax.bitwise_and(p, 31)
            rl = rc * _CHUNK + i
            fxs = lax.shift_right_arithmetic(j - 1, 1)
            x0 = jnp.clip(fxs, 0, _GW - 1)
            x1 = jnp.clip(fxs + 1, 0, _GW - 1)
            colv = j * 16 + lax.iota(i32, 16)
            wx1 = (colv.astype(f32) + 0.5) * (1.0 / 32.0) - 0.5 - fxs.astype(f32)
            g = guide_v[pl.ds(p * 16, 16)]
            t = jnp.clip(g, 0.0, 1.0) * float(_GD) - 0.5
            ti = t.astype(i32)
            fzv = jnp.where(t < ti.astype(f32), ti - 1, ti)
            wz1 = t - fzv.astype(f32)
            z0 = jnp.clip(fzv, 0, _GD - 1)
            z1 = jnp.clip(fzv + 1, 0, _GD - 1)

            rbase = rl * _RROW
            wxs = ((1.0 - wx1), wx1)
            xbs = (rbase + x0 * _GD, rbase + x1 * _GD)
            wzs = ((1.0 - wz1), wz1)

            coeff = [None] * _GC
            for b in range(2):
                for d in range(2):
                    w = wxs[b] * wzs[d]
                    base = (z0 if d == 0 else z1) + xbs[b]
                    for c in range(_GC):
                        gv = plsc.load_gather(rp_v, [base + c * (_GW * _GD)])
                        if coeff[c] is None:
                            coeff[c] = w * gv
                        else:
                            coeff[c] = coeff[c] + w * gv

            for co in range(_COUT):
                res = coeff[(_CIN + 1) * co + _CIN]
                for ci in range(_CIN):
                    imgv = img_v[pl.ds(ci * _CPIX + p * 16, 16)]
                    res = res + coeff[(_CIN + 1) * co + ci] * imgv
                out_v[pl.ds(co * _CPIX + p * 16, 16)] = res

        for co in range(_COUT):
            pltpu.sync_copy(
                out_v.at[pl.ds(co * _CPIX, _CPIX)],
                out_hbm.at[pl.ds((batch * _COUT + co - batch) * _NPIX + off,
                                 _CPIX)])
        return carry

    lax.fori_loop(0, _ROWS_PER_TEC // _CHUNK, chunk_body, 0)


def _run_sc(rp_flat, guide_flat, img_flat):
    mesh = plsc.VectorSubcoreMesh(core_axis_name="c", subcore_axis_name="s")
    f = functools.partial(
        pl.kernel,
        mesh=mesh,
        compiler_params=pltpu.CompilerParams(needs_layout_passes=False),
        out_type=jax.ShapeDtypeStruct((_B * _COUT * _H * _W,), jnp.float32),
        scratch_types=[
            pltpu.VMEM((_ROWS_PER_TEC * _GD * _GC * _GW,), jnp.float32),
            pltpu.VMEM((_CPIX,), jnp.float32),
            pltpu.VMEM((_CIN * _CPIX,), jnp.float32),
            pltpu.VMEM((_COUT * _CPIX,), jnp.float32),
        ],
    )(_sc_body)
    return f(rp_flat, guide_flat, img_flat)


def kernel(grid_th, guide_th, image_th, W_conv, b_conv):
    image_t = jnp.transpose(image_th, (0, 3, 1, 2))
    grid_t = jnp.transpose(grid_th, (0, 1, 4, 2, 3)).reshape(
        _B, _GH, _GD * _GC * _GW)
    conv_t, rp = _run_conv(image_t, grid_t, W_conv, b_conv)
    guide_flat = guide_th.reshape(-1)
    img_flat = conv_t.reshape(-1)
    out_flat = _run_sc(rp.reshape(-1), guide_flat, img_flat)
    out_t = out_flat.reshape(_B, _COUT, _H, _W)
    return jnp.transpose(out_t, (0, 2, 3, 1))
